# Initial kernel scaffold; baseline (speedup 1.0000x reference)
import functools

import jax
import jax.numpy as jnp
from jax import lax
from jax.experimental import pallas as pl
from jax.experimental.pallas import tpu as pltpu
from jax.experimental.pallas import tpu_sc as plsc

N = 10000
E = 320000
D = 128

NC = 2
NS = 16
L = 16
NW = NC * NS
EPW = E // NW
CHUNK = 80
NCHUNK = EPW // CHUNK
RPT = N // NS


def _dot_t(a, w):
    return lax.dot_general(a, w, (((1,), (1,)), ((), ())),
                           preferred_element_type=jnp.float32)


def _tc_pre_body(x_ref, wl_ref, bl_ref, wr_ref, p_ref, r_ref):
    x = x_ref[...]
    p_ref[...] = _dot_t(x, wl_ref[...])
    r_ref[...] = _dot_t(x, wr_ref[...]) + bl_ref[...]


def _tc_pre(x, Wl, bl, Wr):
    return pl.pallas_call(
        _tc_pre_body,
        out_shape=(jax.ShapeDtypeStruct((N, D), jnp.float32),
                   jax.ShapeDtypeStruct((N, D), jnp.float32)),
    )(x, Wl, bl.reshape(1, D), Wr)


def _tc_mid_body(acc_ref, degp_ref, r1_ref, wl_ref, bl_ref, wr_ref,
                 p2_ref, r2_ref, dinv_ref):
    deg = jnp.sum(degp_ref[...], axis=1, keepdims=True)
    dinv = 1.0 / jnp.maximum(deg, 1.0)
    h = jnp.maximum((acc_ref[0] + acc_ref[1]) * dinv + r1_ref[...], 0.0)
    p2_ref[...] = _dot_t(h, wl_ref[...])
    r2_ref[...] = _dot_t(h, wr_ref[...]) + bl_ref[...]
    dinv_ref[...] = dinv


def _tc_mid(acc, deg_parts_t, r1, Wl, bl, Wr):
    return pl.pallas_call(
        _tc_mid_body,
        out_shape=(jax.ShapeDtypeStruct((N, D), jnp.float32),
                   jax.ShapeDtypeStruct((N, D), jnp.float32),
                   jax.ShapeDtypeStruct((N, 1), jnp.float32)),
    )(acc, deg_parts_t, r1, Wl, bl.reshape(1, D), Wr)


def _tc_post_body(acc_ref, dinv_ref, r2_ref, out_ref):
    out_ref[...] = (acc_ref[0] + acc_ref[1]) * dinv_ref[...] + r2_ref[...]


def _tc_post(acc, dinv, r2):
    return pl.pallas_call(
        _tc_post_body,
        out_shape=jax.ShapeDtypeStruct((N, D), jnp.float32),
    )(acc, dinv, r2)


def _make_seg_kernel(want_deg):
    mesh = plsc.VectorSubcoreMesh(core_axis_name="c", subcore_axis_name="s")

    out_type = [jax.ShapeDtypeStruct((NC, N, D), jnp.float32)]
    scratch = [
        pltpu.VMEM((CHUNK,), jnp.int32),
        pltpu.VMEM((CHUNK,), jnp.int32),
        pltpu.VMEM((CHUNK, D), jnp.float32),
        pltpu.VMEM_SHARED((N, D), jnp.float32),
        pltpu.SemaphoreType.DMA,
    ]
    if want_deg:
        out_type.append(jax.ShapeDtypeStruct((NW, N), jnp.float32))
        scratch.append(pltpu.VMEM((N,), jnp.float32))

    def body(y_hbm, src_hbm, dst_hbm, z2_hbm, z1_hbm, *refs):
        if want_deg:
            (acc_out, deg_out, src_v, dst_v, rows_v, acc_sp, sem,
             deg_v) = refs
        else:
            acc_out, src_v, dst_v, rows_v, acc_sp, sem = refs
        c = lax.axis_index("c")
        s = lax.axis_index("s")
        wid = c * NS + s

        pltpu.sync_copy(z2_hbm.at[pl.ds(s * RPT, RPT)],
                        acc_sp.at[pl.ds(s * RPT, RPT)])
        if want_deg:
            pltpu.sync_copy(z1_hbm, deg_v)
        plsc.subcore_barrier()

        ones = jnp.full((L,), 1.0, jnp.float32)

        def chunk_body(k, carry):
            base = wid * EPW + k * CHUNK
            pltpu.sync_copy(src_hbm.at[pl.ds(base, CHUNK)], src_v)
            pltpu.sync_copy(dst_hbm.at[pl.ds(base, CHUNK)], dst_v)
            pltpu.async_copy(y_hbm.at[src_v], rows_v, sem).wait()
            pltpu.sync_copy(rows_v, acc_sp.at[dst_v], add=True)
            if want_deg:
                for j in range(CHUNK // L):
                    idx = dst_v[pl.ds(j * L, L)]
                    plsc.addupdate_scatter(deg_v, [idx], ones)
            return carry

        lax.fori_loop(0, NCHUNK, chunk_body, 0)

        plsc.subcore_barrier()

        pltpu.sync_copy(acc_sp.at[pl.ds(s * RPT, RPT)],
                        acc_out.at[c].at[pl.ds(s * RPT, RPT)])
        if want_deg:
            pltpu.sync_copy(deg_v, deg_out.at[wid])

    return pl.kernel(
        body,
        out_type=tuple(out_type),
        mesh=mesh,
        scratch_types=scratch,
    )


_seg_with_deg = _make_seg_kernel(True)
_seg_no_deg = _make_seg_kernel(False)


def kernel(x, edge_index, W1l, b1l, W1r, W2l, b2l, W2r):
    src = edge_index[0]
    dst = edge_index[1]
    z2 = jnp.zeros((N, D), jnp.float32)
    z1 = jnp.zeros((N,), jnp.float32)

    p1, r1 = _tc_pre(x, W1l, b1l, W1r)
    acc1, deg_parts = _seg_with_deg(p1, src, dst, z2, z1)
    p2, r2, dinv = _tc_mid(acc1, deg_parts.T, r1, W2l, b2l, W2r)
    (acc2,) = _seg_no_deg(p2, src, dst, z2, z1)
    return _tc_post(acc2, dinv, r2)


# speedup vs baseline: 5.5490x; 5.5490x over previous
import functools

import jax
import jax.numpy as jnp
from jax import lax
from jax.experimental import pallas as pl
from jax.experimental.pallas import tpu as pltpu
from jax.experimental.pallas import tpu_sc as plsc

N = 10000
E = 320000
D = 128

NC = 2
NS = 16
L = 16
NW = NC * NS
EPW = E // NW
CHUNK = 80
NCHUNK = EPW // CHUNK
RPT = 624
RPT_LAST = N - RPT * (NS - 1)


def _dot_t(a, w):
    return lax.dot_general(a, w, (((1,), (1,)), ((), ())),
                           preferred_element_type=jnp.float32)


def _tc_pre_body(x_ref, wl_ref, bl_ref, wr_ref, p_ref, r_ref):
    x = x_ref[...]
    p_ref[...] = _dot_t(x, wl_ref[...])
    r_ref[...] = _dot_t(x, wr_ref[...]) + bl_ref[...]


def _tc_pre(x, Wl, bl, Wr):
    return pl.pallas_call(
        _tc_pre_body,
        out_shape=(jax.ShapeDtypeStruct((N, D), jnp.float32),
                   jax.ShapeDtypeStruct((N, D), jnp.float32)),
    )(x, Wl, bl.reshape(1, D), Wr)


def _tc_mid_body(acc_ref, degp_ref, r1_ref, wl_ref, bl_ref, wr_ref,
                 p2_ref, r2_ref, dinv_ref):
    deg = jnp.sum(degp_ref[...], axis=1, keepdims=True)
    dinv = 1.0 / jnp.maximum(deg, 1.0)
    h = jnp.maximum((acc_ref[0] + acc_ref[1]) * dinv + r1_ref[...], 0.0)
    p2_ref[...] = _dot_t(h, wl_ref[...])
    r2_ref[...] = _dot_t(h, wr_ref[...]) + bl_ref[...]
    dinv_ref[...] = dinv


def _tc_mid(acc, deg_parts_t, r1, Wl, bl, Wr):
    return pl.pallas_call(
        _tc_mid_body,
        out_shape=(jax.ShapeDtypeStruct((N, D), jnp.float32),
                   jax.ShapeDtypeStruct((N, D), jnp.float32),
                   jax.ShapeDtypeStruct((N, 1), jnp.float32)),
    )(acc, deg_parts_t, r1, Wl, bl.reshape(1, D), Wr)


def _tc_post_body(acc_ref, dinv_ref, r2_ref, out_ref):
    out_ref[...] = (acc_ref[0] + acc_ref[1]) * dinv_ref[...] + r2_ref[...]


def _tc_post(acc, dinv, r2):
    return pl.pallas_call(
        _tc_post_body,
        out_shape=jax.ShapeDtypeStruct((N, D), jnp.float32),
    )(acc, dinv, r2)


def _make_seg_kernel(want_deg):
    mesh = plsc.VectorSubcoreMesh(core_axis_name="c", subcore_axis_name="s")

    out_type = [jax.ShapeDtypeStruct((NC, N, D), jnp.float32)]
    scratch = [
        pltpu.VMEM((CHUNK,), jnp.int32),
        pltpu.VMEM((CHUNK,), jnp.int32),
        pltpu.VMEM((CHUNK, D), jnp.float32),
        pltpu.VMEM_SHARED((N, D), jnp.float32),
        pltpu.SemaphoreType.DMA,
    ]
    if want_deg:
        out_type.append(jax.ShapeDtypeStruct((NW, N), jnp.float32))
        scratch.append(pltpu.VMEM((N,), jnp.float32))

    def body(y_hbm, src_hbm, dst_hbm, z2_hbm, z1_hbm, *refs):
        if want_deg:
            (acc_out, deg_out, src_v, dst_v, rows_v, acc_sp, sem,
             deg_v) = refs
        else:
            acc_out, src_v, dst_v, rows_v, acc_sp, sem = refs
        c = lax.axis_index("c")
        s = lax.axis_index("s")
        wid = c * NS + s

        @pl.when(s < NS - 1)
        def _():
            pltpu.sync_copy(z2_hbm.at[pl.ds(s * RPT, RPT)],
                            acc_sp.at[pl.ds(s * RPT, RPT)])

        @pl.when(s == NS - 1)
        def _():
            pltpu.sync_copy(z2_hbm.at[pl.ds(RPT * (NS - 1), RPT_LAST)],
                            acc_sp.at[pl.ds(RPT * (NS - 1), RPT_LAST)])

        if want_deg:
            pltpu.sync_copy(z1_hbm, deg_v)
        plsc.subcore_barrier()

        ones = jnp.full((L,), 1.0, jnp.float32)

        def chunk_body(k, carry):
            base = wid * EPW + k * CHUNK
            pltpu.sync_copy(src_hbm.at[pl.ds(base, CHUNK)], src_v)
            pltpu.sync_copy(dst_hbm.at[pl.ds(base, CHUNK)], dst_v)
            pltpu.async_copy(y_hbm.at[src_v], rows_v, sem).wait()
            pltpu.sync_copy(rows_v, acc_sp.at[dst_v], add=True)
            if want_deg:
                for j in range(CHUNK // L):
                    idx = dst_v[pl.ds(j * L, L)]
                    plsc.addupdate_scatter(deg_v, [idx], ones)
            return carry

        lax.fori_loop(0, NCHUNK, chunk_body, 0)

        plsc.subcore_barrier()

        @pl.when(s < NS - 1)
        def _():
            pltpu.sync_copy(acc_sp.at[pl.ds(s * RPT, RPT)],
                            acc_out.at[c].at[pl.ds(s * RPT, RPT)])

        @pl.when(s == NS - 1)
        def _():
            pltpu.sync_copy(acc_sp.at[pl.ds(RPT * (NS - 1), RPT_LAST)],
                            acc_out.at[c].at[pl.ds(RPT * (NS - 1), RPT_LAST)])

        if want_deg:
            pltpu.sync_copy(deg_v, deg_out.at[wid])

    return pl.kernel(
        body,
        out_type=tuple(out_type),
        mesh=mesh,
        scratch_types=scratch,
        compiler_params=pltpu.CompilerParams(needs_layout_passes=False),
    )


_seg_with_deg = _make_seg_kernel(True)
_seg_no_deg = _make_seg_kernel(False)


def kernel(x, edge_index, W1l, b1l, W1r, W2l, b2l, W2r):
    src = edge_index[0]
    dst = edge_index[1]
    z2 = jnp.zeros((N, D), jnp.float32)
    z1 = jnp.zeros((N,), jnp.float32)

    p1, r1 = _tc_pre(x, W1l, b1l, W1r)
    acc1, deg_parts = _seg_with_deg(p1, src, dst, z2, z1)
    p2, r2, dinv = _tc_mid(acc1, deg_parts.T, r1, W2l, b2l, W2r)
    (acc2,) = _seg_no_deg(p2, src, dst, z2, z1)
    return _tc_post(acc2, dinv, r2)


# speedup vs baseline: 14.1441x; 2.5490x over previous
import functools

import jax
import jax.numpy as jnp
from jax import lax
from jax.experimental import pallas as pl
from jax.experimental.pallas import tpu as pltpu
from jax.experimental.pallas import tpu_sc as plsc

N = 10000
E = 320000
D = 128

NC = 2
NS = 16
L = 16
NW = NC * NS
EPW = E // NW
CHUNK = 80
NCHUNK = EPW // CHUNK
RPT = 624
RPT_LAST = N - RPT * (NS - 1)


def _dot_t(a, w):
    return lax.dot_general(a, w, (((1,), (1,)), ((), ())),
                           preferred_element_type=jnp.float32)


def _tc_pre_body(x_ref, wl_ref, bl_ref, wr_ref, p_ref, r_ref):
    x = x_ref[...]
    p_ref[...] = _dot_t(x, wl_ref[...])
    r_ref[...] = _dot_t(x, wr_ref[...]) + bl_ref[...]


def _tc_pre(x, Wl, bl, Wr):
    return pl.pallas_call(
        _tc_pre_body,
        out_shape=(jax.ShapeDtypeStruct((N, D), jnp.float32),
                   jax.ShapeDtypeStruct((N, D), jnp.float32)),
    )(x, Wl, bl.reshape(1, D), Wr)


def _tc_mid_body(acc_ref, degp_ref, r1_ref, wl_ref, bl_ref, wr_ref,
                 p2_ref, r2_ref, dinv_ref):
    ones_w = jnp.ones((NW, 1), jnp.float32)
    deg = lax.dot_general(degp_ref[...], ones_w, (((0,), (0,)), ((), ())),
                          preferred_element_type=jnp.float32)
    dinv = 1.0 / jnp.maximum(deg, 1.0)
    h = jnp.maximum((acc_ref[0] + acc_ref[1]) * dinv + r1_ref[...], 0.0)
    p2_ref[...] = _dot_t(h, wl_ref[...])
    r2_ref[...] = _dot_t(h, wr_ref[...]) + bl_ref[...]
    dinv_ref[...] = dinv


def _tc_mid(acc, deg_parts_t, r1, Wl, bl, Wr):
    return pl.pallas_call(
        _tc_mid_body,
        out_shape=(jax.ShapeDtypeStruct((N, D), jnp.float32),
                   jax.ShapeDtypeStruct((N, D), jnp.float32),
                   jax.ShapeDtypeStruct((N, 1), jnp.float32)),
    )(acc, deg_parts_t, r1, Wl, bl.reshape(1, D), Wr)


def _tc_post_body(acc_ref, dinv_ref, r2_ref, out_ref):
    out_ref[...] = (acc_ref[0] + acc_ref[1]) * dinv_ref[...] + r2_ref[...]


def _tc_post(acc, dinv, r2):
    return pl.pallas_call(
        _tc_post_body,
        out_shape=jax.ShapeDtypeStruct((N, D), jnp.float32),
    )(acc, dinv, r2)


def _make_seg_kernel(want_deg):
    mesh = plsc.VectorSubcoreMesh(core_axis_name="c", subcore_axis_name="s")

    out_type = [jax.ShapeDtypeStruct((NC, N, D), jnp.float32)]
    scratch = [
        pltpu.VMEM((4, 1, CHUNK), jnp.int32),
        pltpu.VMEM((4, 1, CHUNK), jnp.int32),
        pltpu.VMEM((3, CHUNK, D), jnp.float32),
        pltpu.VMEM_SHARED((N, D), jnp.float32),
    ] + [pltpu.SemaphoreType.DMA] * 14
    if want_deg:
        out_type.append(jax.ShapeDtypeStruct((NW, N), jnp.float32))
        scratch.append(pltpu.VMEM((N,), jnp.float32))

    def body(y_hbm, src_hbm, dst_hbm, z2_hbm, z1_hbm, *refs):
        if want_deg:
            (acc_out, deg_out, src_v, dst_v, rows_v, acc_sp, g0, g1, g2,
             t0, t1, t2, i0, i1, i2, i3, d0, d1, d2, d3, deg_v) = refs
        else:
            (acc_out, src_v, dst_v, rows_v, acc_sp, g0, g1, g2,
             t0, t1, t2, i0, i1, i2, i3, d0, d1, d2, d3) = refs
        gsems = (g0, g1, g2)
        ssems = (t0, t1, t2)
        isems = (i0, i1, i2, i3)
        dsems = (d0, d1, d2, d3)
        c = lax.axis_index("c")
        s = lax.axis_index("s")
        wid = c * NS + s

        @pl.when(s < NS - 1)
        def _():
            pltpu.sync_copy(z2_hbm.at[pl.ds(s * RPT, RPT)],
                            acc_sp.at[pl.ds(s * RPT, RPT)])

        @pl.when(s == NS - 1)
        def _():
            pltpu.sync_copy(z2_hbm.at[pl.ds(RPT * (NS - 1), RPT_LAST)],
                            acc_sp.at[pl.ds(RPT * (NS - 1), RPT_LAST)])

        if want_deg:
            pltpu.sync_copy(z1_hbm, deg_v)
        plsc.subcore_barrier()

        ones = jnp.full((L,), 1.0, jnp.float32)

        def idx_start(k, q):
            pltpu.async_copy(src_hbm.at[wid].at[pl.ds(k, 1)], src_v.at[q],
                             isems[q])

        def idx_wait(k, q):
            pltpu.make_async_copy(src_hbm.at[wid].at[pl.ds(k, 1)],
                                  src_v.at[q], isems[q]).wait()

        def didx_start(k, q):
            pltpu.async_copy(dst_hbm.at[wid].at[pl.ds(k, 1)], dst_v.at[q],
                             dsems[q])

        def didx_wait(k, q):
            pltpu.make_async_copy(dst_hbm.at[wid].at[pl.ds(k, 1)],
                                  dst_v.at[q], dsems[q]).wait()

        def gather_start(q, b):
            pltpu.async_copy(y_hbm.at[src_v.at[q].at[0]], rows_v.at[b],
                             gsems[b])

        def gather_wait(q, b):
            pltpu.make_async_copy(y_hbm.at[src_v.at[q].at[0]], rows_v.at[b],
                                  gsems[b]).wait()

        def scatter_start(q, b):
            pltpu.async_copy(rows_v.at[b], acc_sp.at[dst_v.at[q].at[0]],
                             ssems[b], add=True)

        def scatter_wait(q, b):
            pltpu.make_async_copy(rows_v.at[b], acc_sp.at[dst_v.at[q].at[0]],
                                  ssems[b]).wait()

        def deg_update(q):
            if want_deg:
                for j in range(CHUNK // L):
                    idx = dst_v[q, 0, pl.ds(j * L, L)]
                    plsc.addupdate_scatter(deg_v, [idx], ones)

        def step(k, b, q, first=False, static=False):
            gather_wait(q, b)

            def _idx_next():
                idx_start(k + 4, q)

            if static:
                if k + 4 < NCHUNK:
                    _idx_next()
            else:
                pl.when(k + 4 < NCHUNK)(_idx_next)

            didx_wait(k, q)
            scatter_start(q, b)
            deg_update(q)

            def _launch_next():
                if not first:
                    scatter_wait((q + 3) % 4, (b + 2) % 3)

                    def _didx_next():
                        didx_start(k + 3, (q + 3) % 4)

                    if static:
                        if k + 3 < NCHUNK:
                            _didx_next()
                    else:
                        pl.when(k + 3 < NCHUNK)(_didx_next)
                idx_wait(k + 2, (q + 2) % 4)
                gather_start((q + 2) % 4, (b + 2) % 3)

            if static:
                if k + 2 < NCHUNK:
                    _launch_next()
            else:
                pl.when(k + 2 < NCHUNK)(_launch_next)

        for k in range(4):
            idx_start(k, k)
        for k in range(3):
            didx_start(k, k)
        idx_wait(0, 0)
        gather_start(0, 0)
        idx_wait(1, 1)
        gather_start(1, 1)
        step(0, 0, 0, first=True, static=True)
        didx_start(3, 3)
        step(1, 1, 1, static=True)

        def loop_body(i, carry):
            k12 = 2 + 12 * i
            for j in range(12):
                step(k12 + j, (2 + j) % 3, (2 + j) % 4)
            return carry

        n_main = (NCHUNK - 2 - 3) // 12
        lax.fori_loop(0, n_main, loop_body, 0)
        for k in range(2 + 12 * n_main, NCHUNK):
            step(k, k % 3, k % 4, static=True)
        for k in range(NCHUNK - 3, NCHUNK):
            scatter_wait(k % 4, k % 3)

        plsc.subcore_barrier()

        @pl.when(s < NS - 1)
        def _():
            pltpu.sync_copy(acc_sp.at[pl.ds(s * RPT, RPT)],
                            acc_out.at[c].at[pl.ds(s * RPT, RPT)])

        @pl.when(s == NS - 1)
        def _():
            pltpu.sync_copy(acc_sp.at[pl.ds(RPT * (NS - 1), RPT_LAST)],
                            acc_out.at[c].at[pl.ds(RPT * (NS - 1), RPT_LAST)])

        if want_deg:
            pltpu.sync_copy(deg_v, deg_out.at[wid])

    return pl.kernel(
        body,
        out_type=tuple(out_type),
        mesh=mesh,
        scratch_types=scratch,
        compiler_params=pltpu.CompilerParams(needs_layout_passes=False),
    )


_seg_with_deg = _make_seg_kernel(True)
_seg_no_deg = _make_seg_kernel(False)


def kernel(x, edge_index, W1l, b1l, W1r, W2l, b2l, W2r):
    src = edge_index[0].reshape(NW, NCHUNK, CHUNK)
    dst = edge_index[1].reshape(NW, NCHUNK, CHUNK)
    z2 = jnp.zeros((N, D), jnp.float32)
    z1 = jnp.zeros((N,), jnp.float32)

    p1, r1 = _tc_pre(x, W1l, b1l, W1r)
    acc1, deg_parts = _seg_with_deg(p1, src, dst, z2, z1)
    p2, r2, dinv = _tc_mid(acc1, deg_parts, r1, W2l, b2l, W2r)
    (acc2,) = _seg_no_deg(p2, src, dst, z2, z1)
    return _tc_post(acc2, dinv, r2)
